# baseline (speedup 1.0000x reference)
import functools

import jax
import jax.numpy as jnp
from jax import lax
from jax.experimental import pallas as pl
from jax.experimental.pallas import tpu as pltpu
from jax.experimental.pallas import tpu_sc as plsc

N_POS = 1048576
HASH_BITS = 21
MASK = (1 << HASH_BITS) - 1
P2 = 2654435761
NC, NS = 2, 16
NW = NC * NS
B = 256
N_PER_TILE = N_POS // NW
N_CHUNKS = N_PER_TILE // B
NBLK = 1 << (HASH_BITS - 7)
BLK_PER_W = NBLK // NW
TBLK = 64

GRID_NG = (129, 65, 33)
FILL_BATCH = 512


def _batches(ng):
    ncells = ng * ng
    full = ncells // FILL_BATCH
    tail = ncells - full * FILL_BATCH
    tail_pad = -(-tail // 16) * 16
    out = [(i * FILL_BATCH, FILL_BATCH) for i in range(full)]
    if tail_pad:
        out.append((full * FILL_BATCH, tail_pad))
    return out


GRID_BATCHES = [_batches(ng) for ng in GRID_NG]
GRID_WORDS = [(bs[-1][0] + bs[-1][1]) * 4 for bs in GRID_BATCHES]
TAIL_SIZES = sorted({n * 4 for bs in GRID_BATCHES for _, n in bs if n != FILL_BATCH})

_DNUMS = lax.GatherDimensionNumbers(
    offset_dims=(), collapsed_slice_dims=(0,), start_index_map=(0,))


def _take16(x, idx):
    return lax.gather(x, idx[:, None], _DNUMS, (1,),
                      mode=lax.GatherScatterMode.PROMISE_IN_BOUNDS)


def _rsqrt(v):
    i = lax.bitcast_convert_type(v, jnp.int32)
    y = lax.bitcast_convert_type(jnp.int32(0x5F3759DF) - (i >> 1), jnp.float32)
    y = y * (1.5 - (0.5 * v) * y * y)
    return y


_mesh = plsc.VectorSubcoreMesh(core_axis_name="c", subcore_axis_name="s")
_params = pltpu.CompilerParams(
    needs_layout_passes=False, use_tc_tiling_on_sc=False)


@functools.partial(
    pl.kernel,
    out_type=jax.ShapeDtypeStruct((1 << (HASH_BITS - 1), 8), jnp.float32),
    mesh=_mesh,
    compiler_params=_params,
    scratch_types=[
        pltpu.VMEM((TBLK * 512,), jnp.float32),
        pltpu.VMEM((TBLK * 64, 8), jnp.float32),
    ],
)
def _transpose(x0, o0, in_v, out_v):
    wid = lax.axis_index("s") * NC + lax.axis_index("c")

    iota = lax.broadcasted_iota(jnp.int32, (16,), 0)
    lane4 = iota & 3
    grp4 = iota >> 2
    grp8 = iota >> 3
    lane8 = iota & 7
    gpat = lane4 * 128 + grp4

    def mc_body(m, _):
        blk0 = wid * BLK_PER_W + m * TBLK
        pltpu.sync_copy(x0.at[pl.ds(blk0 * 512, TBLK * 512)], in_v)

        def blk_body(b):
            for v in range(32):
                vec = plsc.load_gather(in_v, [b * 512 + 4 * v + gpat])
                plsc.store_scatter(out_v, [b * 64 + 2 * v + grp8, lane8], vec)

        plsc.parallel_loop(0, TBLK)(blk_body)
        pltpu.sync_copy(out_v, o0.at[pl.ds(blk0 * 64, TBLK * 64)])
        return 0

    lax.fori_loop(0, BLK_PER_W // TBLK, mc_body, 0)


@functools.partial(
    pl.kernel,
    out_type=jax.ShapeDtypeStruct((N_POS * 16,), jnp.float32),
    mesh=_mesh,
    compiler_params=_params,
    scratch_types=(
        [pltpu.VMEM((2 * B,), jnp.float32) for _ in range(2)]
        + [pltpu.VMEM((4 * B,), jnp.int32) for _ in range(2)]
        + [pltpu.VMEM((4 * B, 8), jnp.float32) for _ in range(2)]
        + [pltpu.VMEM((16 * B,), jnp.float32) for _ in range(2)]
        + [pltpu.VMEM((16,), jnp.float32)]
        + [pltpu.VMEM((w,), jnp.float32) for w in GRID_WORDS]
        + [pltpu.VMEM((FILL_BATCH * 4,), jnp.int32) for _ in range(2)]
        + [pltpu.VMEM((w,), jnp.int32) for w in TAIL_SIZES]
        + [pltpu.SemaphoreType.DMA for _ in range(9)]
    ),
)
def _mrhg2d(pos_hbm, t8, x1, x2, x3, lw_hbm, out_hbm,
            pv0, pv1, i0a, i0b, r0a, r0b, ov0, ov1, lw_v,
            g1, g2, g3, fidxa, fidxb, *rest):
    fidx2 = (fidxa, fidxb)
    tails = rest[:len(TAIL_SIZES)]
    ts0, ts1, ps0, ps1, os0, os1, fsa, fsb, fsem = rest[len(TAIL_SIZES):]
    fsem2 = (fsa, fsb)
    xs = (x1, x2, x3)
    grids = (g1, g2, g3)
    posv = (pv0, pv1)
    idx0 = (i0a, i0b)
    rows0 = (r0a, r0b)
    outv = (ov0, ov1)
    tsem = (ts0, ts1)
    psem = (ps0, ps1)
    osem = (os0, os1)
    wid = lax.axis_index("s") * NC + lax.axis_index("c")
    tile_base = wid * N_PER_TILE

    pltpu.sync_copy(lw_hbm, lw_v)
    lw_vec = lw_v[...]

    iota = lax.broadcasted_iota(jnp.int32, (16,), 0)
    lane4 = iota & 3
    grp4 = iota >> 2
    rot1 = (iota & ~3) | ((iota + 1) & 3)
    rot2 = (iota & ~3) | ((iota + 2) & 3)
    lane4p4 = lane4 + 4
    lane128 = lane4 * 128
    outpat = grp4 * 16 + lane4
    lw_splat = [_take16(lw_vec, jnp.full((16,), l, jnp.int32)) for l in range(4)]

    fill_plan = []
    nfull = 0
    for g in range(3):
        for off, n in GRID_BATCHES[g]:
            if n == FILL_BATCH:
                fill_plan.append((g, off, n, fidx2[nfull % 2], fsem2[nfull % 2]))
                nfull += 1
            else:
                fill_plan.append((g, off, n, tails[TAIL_SIZES.index(n * 4)],
                                  fsem))
    pending = []
    for g, off, n, buf, sem in fill_plan:
        ng = GRID_NG[g]
        inv_ng = jnp.float32(1.0 / ng)
        for q in list(pending):
            if q[3] is buf:
                pltpu.make_async_copy(xs[q[0]].at[q[3]],
                                      grids[q[0]].at[pl.ds(q[1] * 4, q[2] * 4)],
                                      q[4]).wait()
                pending.remove(q)

        def fill_gen(j, off=off, ng=ng, inv_ng=inv_ng, buf=buf):
            k = off + j * 16 + iota
            gx = (k.astype(jnp.float32) * inv_ng + 0.004).astype(jnp.int32)
            gy = k - gx * ng
            h = ((gx.astype(jnp.uint32)
                  ^ (gy.astype(jnp.uint32) * jnp.uint32(P2)))
                 & jnp.uint32(MASK)).astype(jnp.int32)
            base = ((h >> 7) << 9) + (h & 127)
            for q in range(4):
                rep = _take16(base, 4 * q + grp4)
                buf[pl.ds((j * 4 + q) * 16, 16)] = rep + lane128

        plsc.parallel_loop(0, n // 16)(fill_gen)
        pltpu.async_copy(xs[g].at[buf], grids[g].at[pl.ds(off * 4, n * 4)], sem)
        pending.append((g, off, n, buf, sem))
    for q in pending:
        pltpu.make_async_copy(xs[q[0]].at[q[3]],
                              grids[q[0]].at[pl.ds(q[1] * 4, q[2] * 4)],
                              q[4]).wait()

    def pos_slice(c):
        return pos_hbm.at[pl.ds(2 * (tile_base + c * B), 2 * B)]

    def out_slice(c):
        return out_hbm.at[pl.ds(16 * (tile_base + c * B), 16 * B)]

    def do_hash(c, pb):
        def hash_body(j):
            xoff = (j >> 3) * 256 + (j & 7) * 16 + iota
            px = plsc.load_gather(posv[pb], [xoff])
            py = plsc.load_gather(posv[pb], [xoff + 128])
            ixu = (px * 0.25).astype(jnp.int32).astype(jnp.uint32)
            iyu = (py * 0.25).astype(jnp.int32).astype(jnp.uint32)
            t = iyu * jnp.uint32(P2)
            tn = t + jnp.uint32(P2)
            ixu1 = ixu + jnp.uint32(1)
            h00 = ((ixu ^ t) & jnp.uint32(MASK)).astype(jnp.int32)
            h10 = ((ixu1 ^ t) & jnp.uint32(MASK)).astype(jnp.int32)
            h01 = ((ixu ^ tn) & jnp.uint32(MASK)).astype(jnp.int32)
            h11 = ((ixu1 ^ tn) & jnp.uint32(MASK)).astype(jnp.int32)
            idx0[pb][pl.ds(0 * B + j * 16, 16)] = h00 >> 1
            idx0[pb][pl.ds(1 * B + j * 16, 16)] = h10 >> 1
            idx0[pb][pl.ds(2 * B + j * 16, 16)] = h01 >> 1
            idx0[pb][pl.ds(3 * B + j * 16, 16)] = h11 >> 1

        plsc.parallel_loop(0, B // 16, unroll=2)(hash_body)

    def fire_gather(b):
        pltpu.async_copy(t8.at[idx0[b]], rows0[b], tsem[b])

    def wait_gather(b):
        pltpu.make_async_copy(t8.at[idx0[b]], rows0[b], tsem[b]).wait()

    def do_blend(c, b):
        def blend_one(j):
            row4 = j * 4 + grp4
            xoff = (j >> 5) * 256 + (j & 31) * 4 + grp4
            px = plsc.load_gather(posv[b], [xoff])
            py = plsc.load_gather(posv[b], [xoff + 128])

            def finish(l, acc):
                s1 = acc + _take16(acc, rot1)
                s2 = s1 + _take16(s1, rot2)
                mu = s2 * 0.25
                d = acc - mu
                sq = d * d
                v1 = sq + _take16(sq, rot1)
                v2 = v1 + _take16(v1, rot2)
                inv = _rsqrt(v2 * 0.25 + 1e-5)
                plsc.store_scatter(outv[b], [j * 64 + l * 4 + outpat],
                                   d * inv * lw_splat[l])

            sx = px * 0.25
            sy = py * 0.25
            ixi = sx.astype(jnp.int32)
            iyi = sy.astype(jnp.int32)
            fx = sx - ixi.astype(jnp.float32)
            fy = sy - iyi.astype(jnp.float32)
            gx = 1.0 - fx
            gy = 1.0 - fy
            s = ((ixi ^ iyi) & 1) << 2
            c00 = s + lane4
            c10 = lane4p4 - s
            rv = rows0[b]
            f00 = plsc.load_gather(rv, [0 * B + row4, c00])
            f10 = plsc.load_gather(rv, [1 * B + row4, c10])
            f01 = plsc.load_gather(rv, [2 * B + row4, c10])
            f11 = plsc.load_gather(rv, [3 * B + row4, c00])
            finish(0, (f00 * (gx * gy) + f10 * (fx * gy)
                       + f01 * (gx * fy) + f11 * (fx * fy)))

            for g in range(3):
                ng = GRID_NG[g]
                inv_cs = 0.25 / (2 << g)
                sx = px * inv_cs
                sy = py * inv_cs
                ixi = sx.astype(jnp.int32)
                iyi = sy.astype(jnp.int32)
                fx = sx - ixi.astype(jnp.float32)
                fy = sy - iyi.astype(jnp.float32)
                gx = 1.0 - fx
                gy = 1.0 - fy
                c00 = ((ixi * ng + iyi) << 2) + lane4
                c01 = c00 + 4
                c10 = c00 + 4 * ng
                c11 = c10 + 4
                gr = grids[g]
                f00 = plsc.load_gather(gr, [c00])
                f10 = plsc.load_gather(gr, [c10])
                f01 = plsc.load_gather(gr, [c01])
                f11 = plsc.load_gather(gr, [c11])
                finish(g + 1, (f00 * (gx * gy) + f10 * (fx * gy)
                               + f01 * (gx * fy) + f11 * (fx * fy)))

        plsc.parallel_loop(0, B // 4, unroll=8)(blend_one)

    pltpu.sync_copy(pos_slice(0), posv[0])
    do_hash(0, 0)
    fire_gather(0)
    pltpu.async_copy(pos_slice(1), posv[1], psem[1])

    def pair_body(p, _):
        for b in (0, 1):
            i = 2 * p + b

            @pl.when(i + 1 < N_CHUNKS)
            def _(b=b, i=i):
                pltpu.make_async_copy(pos_slice(0), posv[b ^ 1],
                                      psem[b ^ 1]).wait()
                do_hash(i + 1, b ^ 1)
                fire_gather(b ^ 1)

            wait_gather(b)

            @pl.when(i >= 2)
            def _(b=b, i=i):
                pltpu.make_async_copy(outv[b], out_slice(0), osem[b]).wait()

            do_blend(i, b)
            pltpu.async_copy(outv[b], out_slice(i), osem[b])

            @pl.when(i + 2 < N_CHUNKS)
            def _(b=b, i=i):
                pltpu.async_copy(pos_slice(i + 2), posv[b], psem[b])
        return 0

    lax.fori_loop(0, N_CHUNKS // 2, pair_body, 0)
    pltpu.make_async_copy(outv[0], out_slice(0), osem[0]).wait()
    pltpu.make_async_copy(outv[1], out_slice(0), osem[1]).wait()


def kernel(positions, table0, table1, table2, table3, level_weights):
    xs = [t.reshape(NBLK, 128, 4).transpose(0, 2, 1).reshape(-1)
          for t in (table0, table1, table2, table3)]
    t8_0 = _transpose(xs[0])
    pos1d = positions.reshape(N_POS // 128, 128, 2).transpose(0, 2, 1).reshape(-1)
    lw16 = jnp.zeros((16,), jnp.float32).at[:4].set(level_weights)
    out = _mrhg2d(pos1d, t8_0, xs[1], xs[2], xs[3], lw16)
    return out.reshape(N_POS, 16)


# speedup vs baseline: 1.1722x; 1.1722x over previous
import functools

import jax
import jax.numpy as jnp
from jax import lax
from jax.experimental import pallas as pl
from jax.experimental.pallas import tpu as pltpu
from jax.experimental.pallas import tpu_sc as plsc

N_POS = 1048576
HASH_BITS = 21
MASK = (1 << HASH_BITS) - 1
P2 = 2654435761
NC, NS = 2, 16
NW = NC * NS
B = 256
N_PER_TILE = N_POS // NW
N_CHUNKS = N_PER_TILE // B
NBLK = 1 << (HASH_BITS - 7)
BLK_PER_W = NBLK // NW
TBLK = 64

GRID_NG = (65, 33)
FILL_BATCH = 512


def _batches(ng):
    ncells = ng * ng
    full = ncells // FILL_BATCH
    tail = ncells - full * FILL_BATCH
    tail_pad = -(-tail // 16) * 16
    out = [(i * FILL_BATCH, FILL_BATCH) for i in range(full)]
    if tail_pad:
        out.append((full * FILL_BATCH, tail_pad))
    return out


GRID_BATCHES = [_batches(ng) for ng in GRID_NG]
GRID_WORDS = [(bs[-1][0] + bs[-1][1]) * 4 for bs in GRID_BATCHES]
TAIL_SIZES = sorted({n * 4 for bs in GRID_BATCHES for _, n in bs if n != FILL_BATCH})

_DNUMS = lax.GatherDimensionNumbers(
    offset_dims=(), collapsed_slice_dims=(0,), start_index_map=(0,))


def _take16(x, idx):
    return lax.gather(x, idx[:, None], _DNUMS, (1,),
                      mode=lax.GatherScatterMode.PROMISE_IN_BOUNDS)


def _rsqrt(v):
    i = lax.bitcast_convert_type(v, jnp.int32)
    y = lax.bitcast_convert_type(jnp.int32(0x5F3759DF) - (i >> 1), jnp.float32)
    y = y * (1.5 - (0.5 * v) * y * y)
    return y


_mesh = plsc.VectorSubcoreMesh(core_axis_name="c", subcore_axis_name="s")
_params = pltpu.CompilerParams(
    needs_layout_passes=False, use_tc_tiling_on_sc=False)


NG0 = 257
L0_PER_SUB = 4160
NG1 = 129
L1_PER_SUB = 1056


@functools.partial(
    pl.kernel,
    out_type=jax.ShapeDtypeStruct((N_POS * 16,), jnp.float32),
    mesh=_mesh,
    compiler_params=_params,
    scratch_types=(
        [pltpu.VMEM((2 * B,), jnp.float32) for _ in range(2)]
        + [pltpu.VMEM((4 * B,), jnp.int32) for _ in range(4)]
        + [pltpu.VMEM((4 * B, 8), jnp.float32) for _ in range(4)]
        + [pltpu.VMEM((16 * B,), jnp.float32) for _ in range(2)]
        + [pltpu.VMEM((16,), jnp.float32)]
        + [pltpu.VMEM((w,), jnp.float32) for w in GRID_WORDS]
        + [pltpu.VMEM((FILL_BATCH * 4,), jnp.int32) for _ in range(2)]
        + [pltpu.VMEM((w,), jnp.int32) for w in TAIL_SIZES]
        + [pltpu.VMEM((FILL_BATCH * 4,), jnp.float32)]
        + [pltpu.VMEM((FILL_BATCH, 8), jnp.float32)]
        + [pltpu.VMEM_SHARED((NS * L0_PER_SUB, 8), jnp.float32)]
        + [pltpu.VMEM_SHARED((NS * L1_PER_SUB, 8), jnp.float32)]
        + [pltpu.SemaphoreType.DMA for _ in range(9)]
    ),
)
def _mrhg2d(pos_hbm, x0, x1, x2, x3, lw_hbm, out_hbm,
            pv0, pv1, i0a, i0b, i1a, i1b, r0a, r0b, r1a, r1b,
            ov0, ov1, lw_v, g2, g3, fidxa, fidxb, *rest):
    fidx2 = (fidxa, fidxb)
    tails = rest[:len(TAIL_SIZES)]
    (fbuf, stag, g0s, g1s,
     ts0, ts1, ps0, ps1, os0, os1, fsa, fsb, fsem) = rest[len(TAIL_SIZES):]
    fsem2 = (fsa, fsb)
    xs = (x2, x3)
    grids = (g2, g3)
    posv = (pv0, pv1)
    idx0 = (i0a, i0b)
    idx1 = (i1a, i1b)
    rows0 = (r0a, r0b)
    rows1 = (r1a, r1b)
    outv = (ov0, ov1)
    tsem = (ts0, ts1)
    psem = (ps0, ps1)
    osem = (os0, os1)
    wid = lax.axis_index("s") * NC + lax.axis_index("c")
    tile_base = wid * N_PER_TILE

    pltpu.sync_copy(lw_hbm, lw_v)
    lw_vec = lw_v[...]

    iota = lax.broadcasted_iota(jnp.int32, (16,), 0)
    lane4 = iota & 3
    grp4 = iota >> 2
    rot1 = (iota & ~3) | ((iota + 1) & 3)
    rot2 = (iota & ~3) | ((iota + 2) & 3)
    lane4p4 = lane4 + 4
    lane128 = lane4 * 128
    outpat = grp4 * 16 + lane4
    lw_splat = [_take16(lw_vec, jnp.full((16,), l, jnp.int32)) for l in range(4)]

    fill_plan = []
    nfull = 0
    for g in range(len(GRID_NG)):
        for off, n in GRID_BATCHES[g]:
            if n == FILL_BATCH:
                fill_plan.append((g, off, n, fidx2[nfull % 2], fsem2[nfull % 2]))
                nfull += 1
            else:
                fill_plan.append((g, off, n, tails[TAIL_SIZES.index(n * 4)],
                                  fsem))
    pending = []
    for g, off, n, buf, sem in fill_plan:
        ng = GRID_NG[g]
        inv_ng = jnp.float32(1.0 / ng)
        for q in list(pending):
            if q[3] is buf:
                pltpu.make_async_copy(xs[q[0]].at[q[3]],
                                      grids[q[0]].at[pl.ds(q[1] * 4, q[2] * 4)],
                                      q[4]).wait()
                pending.remove(q)

        def fill_gen(j, off=off, ng=ng, inv_ng=inv_ng, buf=buf):
            k = off + j * 16 + iota
            gx0 = (k.astype(jnp.float32) * inv_ng).astype(jnp.int32)
            gy0 = k - gx0 * ng
            gx = gx0 - (gy0 < 0).astype(jnp.int32) + (gy0 >= ng).astype(jnp.int32)
            gy = k - gx * ng
            h = ((gx.astype(jnp.uint32)
                  ^ (gy.astype(jnp.uint32) * jnp.uint32(P2)))
                 & jnp.uint32(MASK)).astype(jnp.int32)
            base = ((h >> 7) << 9) + (h & 127)
            for q in range(4):
                rep = _take16(base, 4 * q + grp4)
                buf[pl.ds((j * 4 + q) * 16, 16)] = rep + lane128

        plsc.parallel_loop(0, n // 16)(fill_gen)
        pltpu.async_copy(xs[g].at[buf], grids[g].at[pl.ds(off * 4, n * 4)], sem)
        pending.append((g, off, n, buf, sem))
    for q in pending:
        pltpu.make_async_copy(xs[q[0]].at[q[3]],
                              grids[q[0]].at[pl.ds(q[1] * 4, q[2] * 4)],
                              q[4]).wait()

    sid = lax.axis_index("s")
    for src, gsp, ngs, per_sub, nb, tail in (
            (x0, g0s, NG0, L0_PER_SUB, 9, 64),
            (x1, g1s, NG1, L1_PER_SUB, 3, 32)):
        inv_ngs = jnp.float32(1.0 / ngs)
        cell0 = sid * per_sub
        for bb in range(nb):
            ncopy = FILL_BATCH if bb < nb - 1 else tail

            def sp_gen(j, bb=bb, ngs=ngs, inv_ngs=inv_ngs, cell0=cell0):
                k = cell0 + bb * FILL_BATCH + j * 16 + iota
                gx0 = (k.astype(jnp.float32) * inv_ngs).astype(jnp.int32)
                gy0 = k - gx0 * ngs
                gx = (gx0 - (gy0 < 0).astype(jnp.int32)
                      + (gy0 >= ngs).astype(jnp.int32))
                gy = k - gx * ngs
                h = ((gx.astype(jnp.uint32)
                      ^ (gy.astype(jnp.uint32) * jnp.uint32(P2)))
                     & jnp.uint32(MASK)).astype(jnp.int32)
                base = ((h >> 7) << 9) + (h & 127)
                for q in range(4):
                    rep = _take16(base, 4 * q + grp4)
                    fidxa[pl.ds((j * 4 + q) * 16, 16)] = rep + lane128

            plsc.parallel_loop(0, FILL_BATCH // 16)(sp_gen)
            pltpu.async_copy(src.at[fidxa], fbuf, fsem)
            pltpu.make_async_copy(src.at[fidxa], fbuf, fsem).wait()

            def sp_expand(j):
                v = fbuf[pl.ds(j * 16, 16)]
                plsc.store_scatter(stag, [(j * 16 + iota) >> 2, lane4], v)

            plsc.parallel_loop(0, FILL_BATCH * 4 // 16)(sp_expand)
            pltpu.sync_copy(stag.at[pl.ds(0, ncopy)],
                            gsp.at[pl.ds(cell0 + bb * FILL_BATCH, ncopy)])
    plsc.subcore_barrier()

    def pos_slice(c):
        return pos_hbm.at[pl.ds(2 * (tile_base + c * B), 2 * B)]

    def out_slice(c):
        return out_hbm.at[pl.ds(16 * (tile_base + c * B), 16 * B)]

    def do_hash(c, pb):
        def hash_body(j):
            xoff = (j >> 3) * 256 + (j & 7) * 16 + iota
            px = plsc.load_gather(posv[pb], [xoff])
            py = plsc.load_gather(posv[pb], [xoff + 128])
            ixi = (px * 0.25).astype(jnp.int32)
            iyi = (py * 0.25).astype(jnp.int32)
            c00 = ixi * NG0 + iyi
            idx0[pb][pl.ds(0 * B + j * 16, 16)] = c00
            idx0[pb][pl.ds(1 * B + j * 16, 16)] = c00 + NG0
            idx0[pb][pl.ds(2 * B + j * 16, 16)] = c00 + 1
            idx0[pb][pl.ds(3 * B + j * 16, 16)] = c00 + NG0 + 1
            c00 = (ixi >> 1) * NG1 + (iyi >> 1)
            idx1[pb][pl.ds(0 * B + j * 16, 16)] = c00
            idx1[pb][pl.ds(1 * B + j * 16, 16)] = c00 + NG1
            idx1[pb][pl.ds(2 * B + j * 16, 16)] = c00 + 1
            idx1[pb][pl.ds(3 * B + j * 16, 16)] = c00 + NG1 + 1

        plsc.parallel_loop(0, B // 16, unroll=2)(hash_body)

    def fire_gather(b):
        pltpu.async_copy(g0s.at[idx0[b]], rows0[b], tsem[b])
        pltpu.async_copy(g1s.at[idx1[b]], rows1[b], tsem[b])

    def wait_gather(b):
        pltpu.make_async_copy(g0s.at[idx0[b]], rows0[b], tsem[b]).wait()
        pltpu.make_async_copy(g1s.at[idx1[b]], rows1[b], tsem[b]).wait()

    def do_blend(c, b):
        def blend_one(j):
            row4 = j * 4 + grp4
            xoff = (j >> 5) * 256 + (j & 31) * 4 + grp4
            px = plsc.load_gather(posv[b], [xoff])
            py = plsc.load_gather(posv[b], [xoff + 128])

            def finish(l, acc):
                s1 = acc + _take16(acc, rot1)
                s2 = s1 + _take16(s1, rot2)
                mu = s2 * 0.25
                d = acc - mu
                sq = d * d
                v1 = sq + _take16(sq, rot1)
                v2 = v1 + _take16(v1, rot2)
                inv = _rsqrt(v2 * 0.25 + 1e-5)
                plsc.store_scatter(outv[b], [j * 64 + l * 4 + outpat],
                                   d * inv * lw_splat[l])

            sx = px * 0.25
            sy = py * 0.25
            ixi = sx.astype(jnp.int32)
            iyi = sy.astype(jnp.int32)
            fx = sx - ixi.astype(jnp.float32)
            fy = sy - iyi.astype(jnp.float32)
            gx = 1.0 - fx
            gy = 1.0 - fy
            rv = rows0[b]
            f00 = plsc.load_gather(rv, [0 * B + row4, lane4])
            f10 = plsc.load_gather(rv, [1 * B + row4, lane4])
            f01 = plsc.load_gather(rv, [2 * B + row4, lane4])
            f11 = plsc.load_gather(rv, [3 * B + row4, lane4])
            finish(0, (f00 * (gx * gy) + f10 * (fx * gy)
                       + f01 * (gx * fy) + f11 * (fx * fy)))

            sx = px * 0.125
            sy = py * 0.125
            ixi = sx.astype(jnp.int32)
            iyi = sy.astype(jnp.int32)
            fx = sx - ixi.astype(jnp.float32)
            fy = sy - iyi.astype(jnp.float32)
            gx = 1.0 - fx
            gy = 1.0 - fy
            rv = rows1[b]
            f00 = plsc.load_gather(rv, [0 * B + row4, lane4])
            f10 = plsc.load_gather(rv, [1 * B + row4, lane4])
            f01 = plsc.load_gather(rv, [2 * B + row4, lane4])
            f11 = plsc.load_gather(rv, [3 * B + row4, lane4])
            finish(1, (f00 * (gx * gy) + f10 * (fx * gy)
                       + f01 * (gx * fy) + f11 * (fx * fy)))

            for g in range(2):
                ng = GRID_NG[g]
                inv_cs = 0.25 / (4 << g)
                sx = px * inv_cs
                sy = py * inv_cs
                ixi = sx.astype(jnp.int32)
                iyi = sy.astype(jnp.int32)
                fx = sx - ixi.astype(jnp.float32)
                fy = sy - iyi.astype(jnp.float32)
                gx = 1.0 - fx
                gy = 1.0 - fy
                c00 = ((ixi * ng + iyi) << 2) + lane4
                c01 = c00 + 4
                c10 = c00 + 4 * ng
                c11 = c10 + 4
                gr = grids[g]
                f00 = plsc.load_gather(gr, [c00])
                f10 = plsc.load_gather(gr, [c10])
                f01 = plsc.load_gather(gr, [c01])
                f11 = plsc.load_gather(gr, [c11])
                finish(g + 2, (f00 * (gx * gy) + f10 * (fx * gy)
                               + f01 * (gx * fy) + f11 * (fx * fy)))

        plsc.parallel_loop(0, B // 4, unroll=4)(blend_one)

    pltpu.sync_copy(pos_slice(0), posv[0])
    do_hash(0, 0)
    fire_gather(0)
    pltpu.async_copy(pos_slice(1), posv[1], psem[1])

    def pair_body(p, _):
        for b in (0, 1):
            i = 2 * p + b

            @pl.when(i + 1 < N_CHUNKS)
            def _(b=b, i=i):
                pltpu.make_async_copy(pos_slice(0), posv[b ^ 1],
                                      psem[b ^ 1]).wait()
                do_hash(i + 1, b ^ 1)
                fire_gather(b ^ 1)

            wait_gather(b)

            @pl.when(i >= 2)
            def _(b=b, i=i):
                pltpu.make_async_copy(outv[b], out_slice(0), osem[b]).wait()

            do_blend(i, b)
            pltpu.async_copy(outv[b], out_slice(i), osem[b])

            @pl.when(i + 2 < N_CHUNKS)
            def _(b=b, i=i):
                pltpu.async_copy(pos_slice(i + 2), posv[b], psem[b])
        return 0

    lax.fori_loop(0, N_CHUNKS // 2, pair_body, 0)
    pltpu.make_async_copy(outv[0], out_slice(0), osem[0]).wait()
    pltpu.make_async_copy(outv[1], out_slice(0), osem[1]).wait()


def kernel(positions, table0, table1, table2, table3, level_weights):
    xs = [t.reshape(NBLK, 128, 4).transpose(0, 2, 1).reshape(-1)
          for t in (table0, table1, table2, table3)]
    pos1d = positions.reshape(N_POS // 128, 128, 2).transpose(0, 2, 1).reshape(-1)
    lw16 = jnp.zeros((16,), jnp.float32).at[:4].set(level_weights)
    out = _mrhg2d(pos1d, xs[0], xs[1], xs[2], xs[3], lw16)
    return out.reshape(N_POS, 16)


# speedup vs baseline: 1.9813x; 1.6903x over previous
import functools

import jax
import jax.numpy as jnp
from jax import lax
from jax.experimental import pallas as pl
from jax.experimental.pallas import tpu as pltpu
from jax.experimental.pallas import tpu_sc as plsc

N_POS = 1048576
HASH_BITS = 21
MASK = (1 << HASH_BITS) - 1
P2 = 2654435761
NC, NS = 2, 16
NW = NC * NS
B = 256
N_PER_TILE = N_POS // NW
N_CHUNKS = N_PER_TILE // B
NBLK = 1 << (HASH_BITS - 7)
BLK_PER_W = NBLK // NW
TBLK = 64

GRID_NG = (65, 33)
FILL_BATCH = 512


def _batches(ng):
    ncells = ng * ng
    full = ncells // FILL_BATCH
    tail = ncells - full * FILL_BATCH
    tail_pad = -(-tail // 16) * 16
    out = [(i * FILL_BATCH, FILL_BATCH) for i in range(full)]
    if tail_pad:
        out.append((full * FILL_BATCH, tail_pad))
    return out


GRID_BATCHES = [_batches(ng) for ng in GRID_NG]
GRID_WORDS = [(bs[-1][0] + bs[-1][1]) * 4 for bs in GRID_BATCHES]
TAIL_SIZES = sorted({n * 4 for bs in GRID_BATCHES for _, n in bs if n != FILL_BATCH})

_DNUMS = lax.GatherDimensionNumbers(
    offset_dims=(), collapsed_slice_dims=(0,), start_index_map=(0,))


def _take16(x, idx):
    return lax.gather(x, idx[:, None], _DNUMS, (1,),
                      mode=lax.GatherScatterMode.PROMISE_IN_BOUNDS)


def _rsqrt(v):
    i = lax.bitcast_convert_type(v, jnp.int32)
    y = lax.bitcast_convert_type(jnp.int32(0x5F3759DF) - (i >> 1), jnp.float32)
    y = y * (1.5 - (0.5 * v) * y * y)
    return y


_mesh = plsc.VectorSubcoreMesh(core_axis_name="c", subcore_axis_name="s")
_params = pltpu.CompilerParams(
    needs_layout_passes=False, use_tc_tiling_on_sc=False)


NG0 = 257
L0_PER_SUB = 4160
NG1 = 129
L1_PER_SUB = 1056


@functools.partial(
    pl.kernel,
    out_type=jax.ShapeDtypeStruct((N_POS * 16,), jnp.float32),
    mesh=_mesh,
    compiler_params=_params,
    scratch_types=(
        [pltpu.VMEM((2 * B,), jnp.float32) for _ in range(2)]
        + [pltpu.VMEM((4 * B,), jnp.int32) for _ in range(4)]
        + [pltpu.VMEM((4 * B, 8), jnp.float32) for _ in range(4)]
        + [pltpu.VMEM((16 * B,), jnp.float32) for _ in range(2)]
        + [pltpu.VMEM((16,), jnp.float32)]
        + [pltpu.VMEM((w,), jnp.float32) for w in GRID_WORDS]
        + [pltpu.VMEM((FILL_BATCH * 4,), jnp.int32) for _ in range(2)]
        + [pltpu.VMEM((w,), jnp.int32) for w in TAIL_SIZES]
        + [pltpu.VMEM((FILL_BATCH * 4,), jnp.float32)]
        + [pltpu.VMEM((FILL_BATCH, 8), jnp.float32)]
        + [pltpu.VMEM_SHARED((NS * L0_PER_SUB, 8), jnp.float32)]
        + [pltpu.VMEM_SHARED((NS * L1_PER_SUB, 8), jnp.float32)]
        + [pltpu.SemaphoreType.DMA for _ in range(9)]
    ),
)
def _mrhg2d(pos_hbm, x0, x1, x2, x3, lw_hbm, out_hbm,
            pv0, pv1, i0a, i0b, i1a, i1b, r0a, r0b, r1a, r1b,
            ov0, ov1, lw_v, g2, g3, fidxa, fidxb, *rest):
    fidx2 = (fidxa, fidxb)
    tails = rest[:len(TAIL_SIZES)]
    (fbuf, stag, g0s, g1s,
     ts0, ts1, ps0, ps1, os0, os1, fsa, fsb, fsem) = rest[len(TAIL_SIZES):]
    fsem2 = (fsa, fsb)
    xs = (x2, x3)
    grids = (g2, g3)
    posv = (pv0, pv1)
    idx0 = (i0a, i0b)
    idx1 = (i1a, i1b)
    rows0 = (r0a, r0b)
    rows1 = (r1a, r1b)
    outv = (ov0, ov1)
    tsem = (ts0, ts1)
    psem = (ps0, ps1)
    osem = (os0, os1)
    wid = lax.axis_index("s") * NC + lax.axis_index("c")
    tile_base = wid * N_PER_TILE

    pltpu.sync_copy(lw_hbm, lw_v)
    lw_vec = lw_v[...]

    iota = lax.broadcasted_iota(jnp.int32, (16,), 0)
    lane4 = iota & 3
    grp4 = iota >> 2
    rot1 = (iota & ~3) | ((iota + 1) & 3)
    rot2 = (iota & ~3) | ((iota + 2) & 3)
    lane4p4 = lane4 + 4
    lane128 = lane4 * 128
    outpat = lane4 * 128 + grp4
    lw_splat = [_take16(lw_vec, jnp.full((16,), l, jnp.int32)) for l in range(4)]

    fill_plan = []
    nfull = 0
    for g in range(len(GRID_NG)):
        for off, n in GRID_BATCHES[g]:
            if n == FILL_BATCH:
                fill_plan.append((g, off, n, fidx2[nfull % 2], fsem2[nfull % 2]))
                nfull += 1
            else:
                fill_plan.append((g, off, n, tails[TAIL_SIZES.index(n * 4)],
                                  fsem))
    pending = []
    for g, off, n, buf, sem in fill_plan:
        ng = GRID_NG[g]
        inv_ng = jnp.float32(1.0 / ng)
        for q in list(pending):
            if q[3] is buf:
                pltpu.make_async_copy(xs[q[0]].at[q[3]],
                                      grids[q[0]].at[pl.ds(q[1] * 4, q[2] * 4)],
                                      q[4]).wait()
                pending.remove(q)

        def fill_gen(j, off=off, ng=ng, inv_ng=inv_ng, buf=buf):
            k = off + j * 16 + iota
            gx0 = (k.astype(jnp.float32) * inv_ng).astype(jnp.int32)
            gy0 = k - gx0 * ng
            gx = gx0 - (gy0 < 0).astype(jnp.int32) + (gy0 >= ng).astype(jnp.int32)
            gy = k - gx * ng
            h = ((gx.astype(jnp.uint32)
                  ^ (gy.astype(jnp.uint32) * jnp.uint32(P2)))
                 & jnp.uint32(MASK)).astype(jnp.int32)
            base = ((h >> 7) << 9) + (h & 127)
            for q in range(4):
                rep = _take16(base, 4 * q + grp4)
                buf[pl.ds((j * 4 + q) * 16, 16)] = rep + lane128

        plsc.parallel_loop(0, n // 16)(fill_gen)
        pltpu.async_copy(xs[g].at[buf], grids[g].at[pl.ds(off * 4, n * 4)], sem)
        pending.append((g, off, n, buf, sem))
    for q in pending:
        pltpu.make_async_copy(xs[q[0]].at[q[3]],
                              grids[q[0]].at[pl.ds(q[1] * 4, q[2] * 4)],
                              q[4]).wait()

    sid = lax.axis_index("s")
    for src, gsp, ngs, per_sub, nb, tail in (
            (x0, g0s, NG0, L0_PER_SUB, 9, 64),
            (x1, g1s, NG1, L1_PER_SUB, 3, 32)):
        inv_ngs = jnp.float32(1.0 / ngs)
        cell0 = sid * per_sub
        for bb in range(nb):
            ncopy = FILL_BATCH if bb < nb - 1 else tail

            def sp_gen(j, bb=bb, ngs=ngs, inv_ngs=inv_ngs, cell0=cell0):
                k = cell0 + bb * FILL_BATCH + j * 16 + iota
                gx0 = (k.astype(jnp.float32) * inv_ngs).astype(jnp.int32)
                gy0 = k - gx0 * ngs
                gx = (gx0 - (gy0 < 0).astype(jnp.int32)
                      + (gy0 >= ngs).astype(jnp.int32))
                gy = k - gx * ngs
                h = ((gx.astype(jnp.uint32)
                      ^ (gy.astype(jnp.uint32) * jnp.uint32(P2)))
                     & jnp.uint32(MASK)).astype(jnp.int32)
                base = ((h >> 7) << 9) + (h & 127)
                for q in range(4):
                    rep = _take16(base, 4 * q + grp4)
                    fidxa[pl.ds((j * 4 + q) * 16, 16)] = rep + lane128

            plsc.parallel_loop(0, FILL_BATCH // 16)(sp_gen)
            pltpu.async_copy(src.at[fidxa], fbuf, fsem)
            pltpu.make_async_copy(src.at[fidxa], fbuf, fsem).wait()

            def sp_expand(j):
                v = fbuf[pl.ds(j * 16, 16)]
                plsc.store_scatter(stag, [(j * 16 + iota) >> 2, lane4], v)

            plsc.parallel_loop(0, FILL_BATCH * 4 // 16)(sp_expand)
            pltpu.sync_copy(stag.at[pl.ds(0, ncopy)],
                            gsp.at[pl.ds(cell0 + bb * FILL_BATCH, ncopy)])
    plsc.subcore_barrier()

    def pos_slice(c):
        return pos_hbm.at[pl.ds(2 * (tile_base + c * B), 2 * B)]

    HALF = N_POS * 8

    def fire_out(c, b):
        off8 = 8 * (tile_base + c * B)
        pltpu.async_copy(outv[b].at[pl.ds(0, 8 * B)],
                         out_hbm.at[pl.ds(off8, 8 * B)], osem[b])
        pltpu.async_copy(outv[b].at[pl.ds(8 * B, 8 * B)],
                         out_hbm.at[pl.ds(HALF + off8, 8 * B)], osem[b])

    def wait_out(b):
        pltpu.make_async_copy(outv[b].at[pl.ds(0, 8 * B)],
                              out_hbm.at[pl.ds(0, 8 * B)], osem[b]).wait()
        pltpu.make_async_copy(outv[b].at[pl.ds(0, 8 * B)],
                              out_hbm.at[pl.ds(0, 8 * B)], osem[b]).wait()

    def do_hash(c, pb):
        def hash_body(j):
            xoff = (j >> 3) * 256 + (j & 7) * 16 + iota
            px = plsc.load_gather(posv[pb], [xoff])
            py = plsc.load_gather(posv[pb], [xoff + 128])
            ixi = (px * 0.25).astype(jnp.int32)
            iyi = (py * 0.25).astype(jnp.int32)
            c00 = ixi * NG0 + iyi
            idx0[pb][pl.ds(0 * B + j * 16, 16)] = c00
            idx0[pb][pl.ds(1 * B + j * 16, 16)] = c00 + NG0
            idx0[pb][pl.ds(2 * B + j * 16, 16)] = c00 + 1
            idx0[pb][pl.ds(3 * B + j * 16, 16)] = c00 + NG0 + 1
            c00 = (ixi >> 1) * NG1 + (iyi >> 1)
            idx1[pb][pl.ds(0 * B + j * 16, 16)] = c00
            idx1[pb][pl.ds(1 * B + j * 16, 16)] = c00 + NG1
            idx1[pb][pl.ds(2 * B + j * 16, 16)] = c00 + 1
            idx1[pb][pl.ds(3 * B + j * 16, 16)] = c00 + NG1 + 1

        plsc.parallel_loop(0, B // 16, unroll=2)(hash_body)

    def fire_gather(b):
        pltpu.async_copy(g0s.at[idx0[b]], rows0[b], tsem[b])
        pltpu.async_copy(g1s.at[idx1[b]], rows1[b], tsem[b])

    def wait_gather(b):
        pltpu.make_async_copy(g0s.at[idx0[b]], rows0[b], tsem[b]).wait()
        pltpu.make_async_copy(g1s.at[idx1[b]], rows1[b], tsem[b]).wait()

    def do_blend(c, b):
        def blend_one(j):
            row4 = j * 4 + grp4
            xoff = (j >> 5) * 256 + (j & 31) * 4 + grp4
            px = plsc.load_gather(posv[b], [xoff])
            py = plsc.load_gather(posv[b], [xoff + 128])

            def finish(l, acc):
                s1 = acc + _take16(acc, rot1)
                s2 = s1 + _take16(s1, rot2)
                mu = s2 * 0.25
                d = acc - mu
                sq = d * d
                v1 = sq + _take16(sq, rot1)
                v2 = v1 + _take16(v1, rot2)
                inv = _rsqrt(v2 * 0.25 + 1e-5)
                oc = (l >> 1) * 2048 + (l & 1) * 512
                plsc.store_scatter(
                    outv[b],
                    [oc + (j >> 5) * 1024 + (j & 31) * 4 + outpat],
                    d * inv * lw_splat[l])

            sx = px * 0.25
            sy = py * 0.25
            ixi = sx.astype(jnp.int32)
            iyi = sy.astype(jnp.int32)
            fx = sx - ixi.astype(jnp.float32)
            fy = sy - iyi.astype(jnp.float32)
            gx = 1.0 - fx
            gy = 1.0 - fy
            rv = rows0[b]
            f00 = plsc.load_gather(rv, [0 * B + row4, lane4])
            f10 = plsc.load_gather(rv, [1 * B + row4, lane4])
            f01 = plsc.load_gather(rv, [2 * B + row4, lane4])
            f11 = plsc.load_gather(rv, [3 * B + row4, lane4])
            finish(0, (f00 * (gx * gy) + f10 * (fx * gy)
                       + f01 * (gx * fy) + f11 * (fx * fy)))

            sx = px * 0.125
            sy = py * 0.125
            ixi = sx.astype(jnp.int32)
            iyi = sy.astype(jnp.int32)
            fx = sx - ixi.astype(jnp.float32)
            fy = sy - iyi.astype(jnp.float32)
            gx = 1.0 - fx
            gy = 1.0 - fy
            rv = rows1[b]
            f00 = plsc.load_gather(rv, [0 * B + row4, lane4])
            f10 = plsc.load_gather(rv, [1 * B + row4, lane4])
            f01 = plsc.load_gather(rv, [2 * B + row4, lane4])
            f11 = plsc.load_gather(rv, [3 * B + row4, lane4])
            finish(1, (f00 * (gx * gy) + f10 * (fx * gy)
                       + f01 * (gx * fy) + f11 * (fx * fy)))

            for g in range(2):
                ng = GRID_NG[g]
                inv_cs = 0.25 / (4 << g)
                sx = px * inv_cs
                sy = py * inv_cs
                ixi = sx.astype(jnp.int32)
                iyi = sy.astype(jnp.int32)
                fx = sx - ixi.astype(jnp.float32)
                fy = sy - iyi.astype(jnp.float32)
                gx = 1.0 - fx
                gy = 1.0 - fy
                c00 = ((ixi * ng + iyi) << 2) + lane4
                c01 = c00 + 4
                c10 = c00 + 4 * ng
                c11 = c10 + 4
                gr = grids[g]
                f00 = plsc.load_gather(gr, [c00])
                f10 = plsc.load_gather(gr, [c10])
                f01 = plsc.load_gather(gr, [c01])
                f11 = plsc.load_gather(gr, [c11])
                finish(g + 2, (f00 * (gx * gy) + f10 * (fx * gy)
                               + f01 * (gx * fy) + f11 * (fx * fy)))

        plsc.parallel_loop(0, B // 4, unroll=4)(blend_one)

    pltpu.sync_copy(pos_slice(0), posv[0])
    do_hash(0, 0)
    fire_gather(0)
    pltpu.async_copy(pos_slice(1), posv[1], psem[1])

    def pair_body(p, _):
        for b in (0, 1):
            i = 2 * p + b

            @pl.when(i + 1 < N_CHUNKS)
            def _(b=b, i=i):
                pltpu.make_async_copy(pos_slice(0), posv[b ^ 1],
                                      psem[b ^ 1]).wait()
                do_hash(i + 1, b ^ 1)
                fire_gather(b ^ 1)

            wait_gather(b)

            @pl.when(i >= 2)
            def _(b=b, i=i):
                wait_out(b)

            do_blend(i, b)
            fire_out(i, b)

            @pl.when(i + 2 < N_CHUNKS)
            def _(b=b, i=i):
                pltpu.async_copy(pos_slice(i + 2), posv[b], psem[b])
        return 0

    lax.fori_loop(0, N_CHUNKS // 2, pair_body, 0)
    wait_out(0)
    wait_out(1)


def kernel(positions, table0, table1, table2, table3, level_weights):
    xs = [t.reshape(NBLK, 128, 4).transpose(0, 2, 1).reshape(-1)
          for t in (table0, table1, table2, table3)]
    pos1d = positions.reshape(N_POS // 128, 128, 2).transpose(0, 2, 1).reshape(-1)
    lw16 = jnp.zeros((16,), jnp.float32).at[:4].set(level_weights)
    out = _mrhg2d(pos1d, xs[0], xs[1], xs[2], xs[3], lw16)
    return (out.reshape(2, N_POS // 128, 8, 128)
            .transpose(1, 3, 0, 2).reshape(N_POS, 16))
